# baseline (speedup 1.0000x reference)
import jax
import jax.numpy as jnp
from jax import lax
from jax.experimental import pallas as pl
from jax.experimental.pallas import tpu as pltpu
from jax.experimental.pallas import tpu_sc as plsc

_CH = 64


def _sc_body(x_hbm, kk_hbm, tok_hbm, out_hbm, kk_v, mask_buf):
    b = lax.axis_index("c")
    t = lax.axis_index("s")

    pltpu.sync_copy(kk_hbm, kk_v.at[pl.ds(0, 16)])
    kk_t = kk_v[pl.ds(t, 16)][0]

    pltpu.sync_copy(tok_hbm, mask_buf.at[0])
    ncol = mask_buf.shape[1] // 16

    def rep_row(i, carry):
        for j in range(ncol):
            sl = pl.ds(j * 16, 16)
            mask_buf[i, sl] = mask_buf[0, sl]
        return carry

    lax.fori_loop(1, _CH, rep_row, 0)

    q = kk_t // _CH
    r = kk_t % _CH

    def copy_chunk(i, carry):
        sl = pl.ds(i * _CH, _CH)
        pltpu.sync_copy(x_hbm.at[b, t, sl], out_hbm.at[b, t, sl])
        return carry

    lax.fori_loop(0, q, copy_chunk, 0)

    qf = q + jnp.where(r > 0, 1, 0).astype(q.dtype)

    def fill_chunk(j, carry):
        pltpu.sync_copy(mask_buf, out_hbm.at[b, t, pl.ds(j * _CH, _CH)])
        return carry

    lax.fori_loop(qf, 16, fill_chunk, 0)

    @pl.when(r > 0)
    def _boundary():
        pltpu.sync_copy(mask_buf, out_hbm.at[b, t, pl.ds(q * _CH, _CH)])

        def copy_row(i, carry):
            n = q * _CH + i
            pltpu.sync_copy(x_hbm.at[b, t, n], out_hbm.at[b, t, n])
            return carry

        lax.fori_loop(0, r, copy_row, 0)


def kernel(x, keep_k, mask_token):
    mesh = plsc.VectorSubcoreMesh(core_axis_name="c", subcore_axis_name="s")
    f = pl.kernel(
        _sc_body,
        out_type=jax.ShapeDtypeStruct(x.shape, x.dtype),
        mesh=mesh,
        scratch_types=[
            pltpu.VMEM((32,), jnp.int32),
            pltpu.VMEM((_CH, x.shape[-1]), jnp.float32),
        ],
    )
    return f(x, keep_k.astype(jnp.int32), mask_token)


# speedup vs baseline: 1.0049x; 1.0049x over previous
import jax
import jax.numpy as jnp
from jax import lax
from jax.experimental import pallas as pl
from jax.experimental.pallas import tpu as pltpu
from jax.experimental.pallas import tpu_sc as plsc

_CH = 64
_NCH = 1024 // _CH


def _sc_body(x_hbm, kk_hbm, tok_hbm, out_hbm, kk_v, mask_buf, sem):
    b = lax.axis_index("c")
    t = lax.axis_index("s")

    pltpu.sync_copy(kk_hbm, kk_v.at[pl.ds(0, 16)])
    kk_t = kk_v[pl.ds(t, 16)][0]

    pltpu.sync_copy(tok_hbm, mask_buf.at[0])
    ncol = mask_buf.shape[1] // 16

    def rep_row(i, carry):
        for j in range(ncol):
            sl = pl.ds(j * 16, 16)
            mask_buf[i, sl] = mask_buf[0, sl]
        return carry

    lax.fori_loop(1, _CH, rep_row, 0)

    for c in range(_NCH):
        lo = c * _CH
        hi = lo + _CH
        sl = pl.ds(lo, _CH)

        @pl.when(kk_t >= hi)
        def _copy():
            pltpu.async_copy(x_hbm.at[b, t, sl], out_hbm.at[b, t, sl], sem)

        @pl.when(kk_t <= lo)
        def _fill():
            pltpu.async_copy(mask_buf, out_hbm.at[b, t, sl], sem)

        @pl.when(jnp.logical_and(kk_t > lo, kk_t < hi))
        def _mixed():
            def copy_row(i, carry):
                pltpu.async_copy(x_hbm.at[b, t, lo + i],
                                 out_hbm.at[b, t, lo + i], sem)
                return carry

            lax.fori_loop(0, kk_t - lo, copy_row, 0)

            def fill_row(i, carry):
                pltpu.async_copy(mask_buf.at[0],
                                 out_hbm.at[b, t, kk_t + i], sem)
                return carry

            lax.fori_loop(0, hi - kk_t, fill_row, 0)

    pltpu.make_async_copy(x_hbm.at[b, t], out_hbm.at[b, t], sem).wait()


def kernel(x, keep_k, mask_token):
    mesh = plsc.VectorSubcoreMesh(core_axis_name="c", subcore_axis_name="s")
    f = pl.kernel(
        _sc_body,
        out_type=jax.ShapeDtypeStruct(x.shape, x.dtype),
        mesh=mesh,
        scratch_types=[
            pltpu.VMEM((32,), jnp.int32),
            pltpu.VMEM((_CH, x.shape[-1]), jnp.float32),
            pltpu.SemaphoreType.DMA,
        ],
    )
    return f(x, keep_k.astype(jnp.int32), mask_token)
